# baseline (speedup 1.0000x reference)
import functools

import jax
import jax.numpy as jnp
from jax import lax
from jax.experimental import pallas as pl
from jax.experimental.pallas import tpu as pltpu
from jax.experimental.pallas import tpu_sc as plsc

HIDDEN = 64
PAD = 128
NC, NS = 2, 16
NW = NC * NS
K = 2
LANES = 16


@functools.partial(jax.jit, static_argnums=(4, 5))
def _embed_pair(sid_flat, ph_flat, stab, ptab, b_dim, l_dim):
    b_per_w = b_dim // NW
    nb = b_per_w // K

    mesh = plsc.VectorSubcoreMesh(core_axis_name="c", subcore_axis_name="s")

    @functools.partial(
        pl.kernel,
        mesh=mesh,
        out_type=[
            jax.ShapeDtypeStruct((b_dim, l_dim, HIDDEN), jnp.float32),
            jax.ShapeDtypeStruct((b_dim, l_dim, HIDDEN), jnp.float32),
        ],
        scratch_types=[
            [pltpu.VMEM((l_dim,), jnp.int32) for _ in range(K)],
            [pltpu.VMEM((l_dim, PAD), jnp.float32) for _ in range(K)],
            [pltpu.VMEM((l_dim, HIDDEN), jnp.float32) for _ in range(K)],
            pltpu.SemaphoreType.DMA,
            pltpu.SemaphoreType.DMA,
            pltpu.SemaphoreType.DMA,
        ],
        compiler_params=pltpu.CompilerParams(use_tc_tiling_on_sc=True),
    )
    def emb(sid_hbm, ph_hbm, stab_hbm, ptab_hbm, out_s, out_p,
            idx_slots, row_slots, cmp_slots, isem, gsem, osem):
        wid = lax.axis_index("s") * NC + lax.axis_index("c")
        b0 = pl.multiple_of(wid * b_per_w, b_per_w)

        def run_table(idx_hbm, tab_hbm, out_hbm):

            def fire_idx(t, j):
                off = pl.multiple_of((b0 + t) * l_dim, 8)
                pltpu.async_copy(
                    idx_hbm.at[pl.ds(off, l_dim)], idx_slots[j], isem)

            def wait_idx(t, j):
                off = pl.multiple_of((b0 + t) * l_dim, 8)
                pltpu.make_async_copy(
                    idx_hbm.at[pl.ds(off, l_dim)], idx_slots[j], isem).wait()

            def fire_gather(t, j):
                pltpu.async_copy(tab_hbm.at[idx_slots[j]], row_slots[j], gsem)

            def wait_gather(t, j):
                pltpu.make_async_copy(
                    tab_hbm.at[idx_slots[j]], row_slots[j], gsem).wait()

            def compact(j):
                def row_copy(r, carry):
                    for c in range(HIDDEN // LANES):
                        cmp_slots[j][r, pl.ds(c * LANES, LANES)] = (
                            row_slots[j][r, pl.ds(c * LANES, LANES)])
                    return carry
                lax.fori_loop(0, l_dim, row_copy, 0)

            def fire_out(t, j):
                pltpu.async_copy(cmp_slots[j], out_hbm.at[b0 + t], osem)

            def wait_out(t, j):
                pltpu.make_async_copy(
                    cmp_slots[j], out_hbm.at[b0 + t], osem).wait()

            for j in range(K):
                fire_idx(j, j)
            for j in range(K):
                wait_idx(j, j)
                fire_gather(j, j)

            def batch(g, carry):
                o0 = g * K
                for j in range(K):
                    wait_gather(o0 + j, j)
                    compact(j)
                    fire_out(o0 + j, j)
                for j in range(K):
                    wait_out(o0 + j, j)
                    fire_idx(o0 + K + j, j)
                for j in range(K):
                    wait_idx(o0 + K + j, j)
                    fire_gather(o0 + K + j, j)
                return carry

            if nb > 1:
                lax.fori_loop(0, nb - 1, batch, 0)
            o0 = (nb - 1) * K
            for j in range(K):
                wait_gather(o0 + j, j)
                compact(j)
                fire_out(o0 + j, j)
            for j in range(K):
                wait_out(o0 + j, j)

        run_table(sid_hbm, stab_hbm, out_s)
        run_table(ph_hbm, ptab_hbm, out_p)

    return tuple(emb(sid_flat, ph_flat, stab, ptab))


def kernel(speaker_id, phone, train, speaker_id_table, phone_table):
    del train
    b_dim, l_dim = speaker_id.shape
    stab = jnp.pad(speaker_id_table, ((0, 0), (0, PAD - HIDDEN)))
    ptab = jnp.pad(phone_table, ((0, 0), (0, PAD - HIDDEN)))
    return _embed_pair(speaker_id.reshape(-1), phone.reshape(-1),
                       stab, ptab, b_dim, l_dim)


# speedup vs baseline: 1.0583x; 1.0583x over previous
import functools

import jax
import jax.numpy as jnp
from jax import lax
from jax.experimental import pallas as pl
from jax.experimental.pallas import tpu as pltpu
from jax.experimental.pallas import tpu_sc as plsc

HIDDEN = 64
NC, NS = 2, 16
NW = NC * NS
K = 4


@functools.partial(jax.jit, static_argnums=(2, 3))
def _embed_one(idx_flat, table, b_dim, l_dim):
    b_per_w = b_dim // NW
    nb = b_per_w // K

    mesh = plsc.VectorSubcoreMesh(core_axis_name="c", subcore_axis_name="s")

    @functools.partial(
        pl.kernel,
        mesh=mesh,
        out_type=jax.ShapeDtypeStruct((b_dim, l_dim, HIDDEN), jnp.float32),
        scratch_types=[
            [pltpu.VMEM((l_dim,), jnp.int32) for _ in range(K)],
            [pltpu.VMEM((l_dim, HIDDEN), jnp.float32) for _ in range(K)],
            pltpu.SemaphoreType.DMA,
            pltpu.SemaphoreType.DMA,
            pltpu.SemaphoreType.DMA,
        ],
        compiler_params=pltpu.CompilerParams(use_tc_tiling_on_sc=False),
    )
    def emb(idx_hbm, tab_hbm, out_hbm, idx_slots, row_slots,
            isem, gsem, osem):
        wid = lax.axis_index("s") * NC + lax.axis_index("c")
        b0 = pl.multiple_of(wid * b_per_w, b_per_w)


        def fire_idx(t, j):
            off = pl.multiple_of((b0 + t) * l_dim, 8)
            pltpu.async_copy(idx_hbm.at[pl.ds(off, l_dim)], idx_slots[j], isem)

        def wait_idx(t, j):
            off = pl.multiple_of((b0 + t) * l_dim, 8)
            pltpu.make_async_copy(
                idx_hbm.at[pl.ds(off, l_dim)], idx_slots[j], isem).wait()

        def fire_gather(t, j):
            pltpu.async_copy(tab_hbm.at[idx_slots[j]], row_slots[j], gsem)

        def wait_gather(t, j):
            pltpu.make_async_copy(
                tab_hbm.at[idx_slots[j]], row_slots[j], gsem).wait()

        def fire_out(t, j):
            pltpu.async_copy(row_slots[j], out_hbm.at[b0 + t], osem)

        def wait_out(t, j):
            pltpu.make_async_copy(
                row_slots[j], out_hbm.at[b0 + t], osem).wait()

        for j in range(K):
            fire_idx(j, j)
        for j in range(K):
            wait_idx(j, j)
            fire_gather(j, j)

        def batch(g, carry):
            o0 = g * K
            for j in range(K):
                wait_gather(o0 + j, j)
                fire_out(o0 + j, j)
            for j in range(K):
                wait_out(o0 + j, j)
                fire_idx(o0 + K + j, j)
            for j in range(K):
                wait_idx(o0 + K + j, j)
                fire_gather(o0 + K + j, j)
            return carry

        if nb > 1:
            lax.fori_loop(0, nb - 1, batch, 0)
        o0 = (nb - 1) * K
        for j in range(K):
            wait_gather(o0 + j, j)
            fire_out(o0 + j, j)
        for j in range(K):
            wait_out(o0 + j, j)

    return emb(idx_flat, table)


def kernel(speaker_id, phone, train, speaker_id_table, phone_table):
    del train
    b_dim, l_dim = speaker_id.shape
    out_s = _embed_one(speaker_id.reshape(-1), speaker_id_table, b_dim, l_dim)
    out_p = _embed_one(phone.reshape(-1), phone_table, b_dim, l_dim)
    return (out_s, out_p)
